# baseline (device time: 84537 ns/iter reference)
import jax
import jax.numpy as jnp
from jax import lax
from jax.experimental import pallas as pl
from jax.experimental.pallas import tpu as pltpu

N_DEV = 4
B, S, H, Dh, Dr = 2, 256, 16, 64, 32
D = 1024
ROWS = B * S
HEADS_PER = H // N_DEV
KBLK = HEADS_PER * Dh
QRBLK = HEADS_PER * Dr
SCALE = (Dh + Dr) ** -0.5


def _dot(a, b):
    return jnp.dot(a, b, preferred_element_type=jnp.float32)


def _dot_t(a, b):
    return lax.dot_general(
        a, b, (((1,), (1,)), ((), ())), preferred_element_type=jnp.float32
    )


def kernel(x, Wdkv, Wuk, Wuv, Wq, Wqr, Wkr, Wo):
    x2 = x.reshape(ROWS, D)

    def body(x_ref, wdkv_ref, wuk_ref, wuv_ref, wq_ref, wqr_ref, wkr_ref,
             wo_ref, out_ref,
             kvp_ref, kv_rx_ref, o_ref, o_rx_ref,
             kv_send_sems, kv_recv_sems, o_send_sems, o_recv_sems):
        p = lax.axis_index("i")
        left = lax.rem(p + N_DEV - 1, N_DEV)
        right = lax.rem(p + 1, N_DEV)

        barrier = pltpu.get_barrier_semaphore()
        for nbr in (left, right):
            pl.semaphore_signal(
                barrier, inc=1, device_id=(nbr,),
                device_id_type=pl.DeviceIdType.MESH,
            )
        pl.semaphore_wait(barrier, 2)

        xv = x_ref[...]
        c = _dot(xv, wdkv_ref[...])

        for j in range(N_DEV):
            kvp_ref[j, :, :KBLK] = _dot(c, wuk_ref[:, j * KBLK:(j + 1) * KBLK])
            kvp_ref[j, :, KBLK:] = _dot(c, wuv_ref[:, j * KBLK:(j + 1) * KBLK])

        for h in range(N_DEV - 1):
            j_send = lax.rem(p - h - 1 + 2 * N_DEV, N_DEV)
            src = kvp_ref.at[j_send] if h == 0 else kv_rx_ref.at[h - 1]
            rdma = pltpu.make_async_remote_copy(
                src_ref=src,
                dst_ref=kv_rx_ref.at[h],
                send_sem=kv_send_sems.at[h],
                recv_sem=kv_recv_sems.at[h],
                device_id=(right,),
                device_id_type=pl.DeviceIdType.MESH,
            )
            rdma.start()
            rdma.wait()
            j_recv = lax.rem(p - h - 2 + 2 * N_DEV, N_DEV)
            kv_rx_ref[h] = kv_rx_ref[h] + kvp_ref[j_recv]

        kv = kv_rx_ref[N_DEV - 2]

        qm = _dot(xv, wq_ref[:, pl.ds(p * KBLK, KBLK)])
        qr = _dot(xv, wqr_ref[:, pl.ds(p * QRBLK, QRBLK)])
        kr = _dot(xv, wkr_ref[...])

        for b in range(B):
            r0 = b * S
            krb = kr[r0:r0 + S, :]
            for hh in range(HEADS_PER):
                q = qm[r0:r0 + S, hh * Dh:(hh + 1) * Dh]
                k = kv[r0:r0 + S, hh * Dh:(hh + 1) * Dh]
                v = kv[r0:r0 + S, KBLK + hh * Dh:KBLK + (hh + 1) * Dh]
                qrh = qr[r0:r0 + S, hh * Dr:(hh + 1) * Dr]
                sc = (_dot_t(q, k) + _dot_t(qrh, krb)) * SCALE
                m = jnp.max(sc, axis=1, keepdims=True)
                e = jnp.exp(sc - m)
                prob = e / jnp.sum(e, axis=1, keepdims=True)
                o_ref[r0:r0 + S, hh * Dh:(hh + 1) * Dh] = _dot(prob, v)

        out_ref[...] = _dot(o_ref[...], wo_ref[pl.ds(p * KBLK, KBLK), :])
        for h in range(N_DEV - 1):
            src = o_ref if h == 0 else o_rx_ref.at[h - 1]
            rdma = pltpu.make_async_remote_copy(
                src_ref=src,
                dst_ref=o_rx_ref.at[h],
                send_sem=o_send_sems.at[h],
                recv_sem=o_recv_sems.at[h],
                device_id=(right,),
                device_id_type=pl.DeviceIdType.MESH,
            )
            rdma.start()
            rdma.wait()
            origin = lax.rem(p - h - 1 + 2 * N_DEV, N_DEV)
            out_ref[...] = out_ref[...] + _dot(
                o_rx_ref[h], wo_ref[pl.ds(origin * KBLK, KBLK), :]
            )

    out = pl.pallas_call(
        body,
        out_shape=jax.ShapeDtypeStruct((ROWS, D), jnp.float32),
        in_specs=[pl.BlockSpec(memory_space=pltpu.VMEM)] * 8,
        out_specs=pl.BlockSpec(memory_space=pltpu.VMEM),
        scratch_shapes=[
            pltpu.VMEM((N_DEV, ROWS, 2 * KBLK), jnp.float32),
            pltpu.VMEM((N_DEV - 1, ROWS, 2 * KBLK), jnp.float32),
            pltpu.VMEM((ROWS, KBLK), jnp.float32),
            pltpu.VMEM((N_DEV - 1, ROWS, KBLK), jnp.float32),
            pltpu.SemaphoreType.DMA((N_DEV - 1,)),
            pltpu.SemaphoreType.DMA((N_DEV - 1,)),
            pltpu.SemaphoreType.DMA((N_DEV - 1,)),
            pltpu.SemaphoreType.DMA((N_DEV - 1,)),
        ],
        compiler_params=pltpu.CompilerParams(collective_id=0),
    )(x2, Wdkv, Wuk, Wuv, Wq, Wqr, Wkr, Wo)
    return out.reshape(B, S, D)


# device time: 46530 ns/iter; 1.8168x vs baseline; 1.8168x over previous
import jax
import jax.numpy as jnp
from jax import lax
from jax.experimental import pallas as pl
from jax.experimental.pallas import tpu as pltpu

N_DEV = 4
B, S, H, Dh, Dr = 2, 256, 16, 64, 32
D = 1024
DC = 64
ROWS = B * S
HEADS_PER = H // N_DEV
KBLK = HEADS_PER * Dh
QRBLK = HEADS_PER * Dr
SCALE = (Dh + Dr) ** -0.5


def _dot(a, b):
    return jnp.dot(a, b, preferred_element_type=jnp.float32)


def _dot_t(a, b):
    return lax.dot_general(
        a, b, (((1,), (1,)), ((), ())), preferred_element_type=jnp.float32
    )


def kernel(x, Wdkv, Wuk, Wuv, Wq, Wqr, Wkr, Wo):
    x2 = x.reshape(ROWS, D)

    def body(x_ref, wdkv_ref, wuk_ref, wuv_ref, wq_ref, wqr_ref, wkr_ref,
             wo_ref, out_ref,
             c_stage_ref, c_gat_ref, wuk_gat_ref, wuv_gat_ref,
             kv_ref, o_ref, o_gat_ref,
             c_ssem, c_rsem, wuk_ssem, wuk_rsem, wuv_ssem, wuv_rsem,
             o_ssem, o_rsem):
        p = lax.axis_index("i")

        barrier = pltpu.get_barrier_semaphore()
        for d in range(1, N_DEV):
            nbr = lax.rem(p + d, N_DEV)
            pl.semaphore_signal(
                barrier, inc=1, device_id=(nbr,),
                device_id_type=pl.DeviceIdType.MESH,
            )
        pl.semaphore_wait(barrier, N_DEV - 1)

        xv = x_ref[...]
        c = _dot(xv, wdkv_ref[...])
        c_stage_ref[...] = c

        sends = []
        for d in range(1, N_DEV):
            q = lax.rem(p + d, N_DEV)
            for src, dst, ssem, rsem in (
                (c_stage_ref, c_gat_ref.at[p], c_ssem.at[q], c_rsem.at[p]),
                (wuk_ref.at[:, pl.ds(q * KBLK, KBLK)], wuk_gat_ref.at[p],
                 wuk_ssem.at[q], wuk_rsem.at[p]),
                (wuv_ref.at[:, pl.ds(q * KBLK, KBLK)], wuv_gat_ref.at[p],
                 wuv_ssem.at[q], wuv_rsem.at[p]),
            ):
                rdma = pltpu.make_async_remote_copy(
                    src_ref=src, dst_ref=dst, send_sem=ssem, recv_sem=rsem,
                    device_id=(q,), device_id_type=pl.DeviceIdType.MESH,
                )
                rdma.start()
                sends.append(rdma)

        qm = _dot(xv, wq_ref[:, pl.ds(p * KBLK, KBLK)])
        qr = _dot(xv, wqr_ref[:, pl.ds(p * QRBLK, QRBLK)])
        kr = _dot(xv, wkr_ref[...])

        kv_ref[:, :KBLK] = _dot(c, wuk_ref[:, pl.ds(p * KBLK, KBLK)])
        kv_ref[:, KBLK:] = _dot(c, wuv_ref[:, pl.ds(p * KBLK, KBLK)])

        for d in (1, 3, 2):
            j = lax.rem(p + d, N_DEV)
            for gat, rsem in ((c_gat_ref, c_rsem), (wuk_gat_ref, wuk_rsem),
                              (wuv_gat_ref, wuv_rsem)):
                pltpu.make_async_remote_copy(
                    src_ref=gat.at[j], dst_ref=gat.at[j],
                    send_sem=rsem.at[j], recv_sem=rsem.at[j],
                    device_id=(j,), device_id_type=pl.DeviceIdType.MESH,
                ).wait_recv()
            kv_ref[:, :KBLK] = kv_ref[:, :KBLK] + _dot(
                c_gat_ref[j], wuk_gat_ref[j])
            kv_ref[:, KBLK:] = kv_ref[:, KBLK:] + _dot(
                c_gat_ref[j], wuv_gat_ref[j])

        for rdma in sends:
            rdma.wait_send()

        kv = kv_ref[...]
        for b in range(B):
            r0 = b * S
            krb = kr[r0:r0 + S, :]
            for hh in range(HEADS_PER):
                q = qm[r0:r0 + S, hh * Dh:(hh + 1) * Dh]
                k = kv[r0:r0 + S, hh * Dh:(hh + 1) * Dh]
                v = kv[r0:r0 + S, KBLK + hh * Dh:KBLK + (hh + 1) * Dh]
                qrh = qr[r0:r0 + S, hh * Dr:(hh + 1) * Dr]
                sc = (_dot_t(q, k) + _dot_t(qrh, krb)) * SCALE
                m = jnp.max(sc, axis=1, keepdims=True)
                e = jnp.exp(sc - m)
                prob = e / jnp.sum(e, axis=1, keepdims=True)
                o_ref[r0:r0 + S, hh * Dh:(hh + 1) * Dh] = _dot(prob, v)

        o_sends = []
        for d in range(1, N_DEV):
            q = lax.rem(p + d, N_DEV)
            rdma = pltpu.make_async_remote_copy(
                src_ref=o_ref, dst_ref=o_gat_ref.at[p],
                send_sem=o_ssem.at[q], recv_sem=o_rsem.at[p],
                device_id=(q,), device_id_type=pl.DeviceIdType.MESH,
            )
            rdma.start()
            o_sends.append(rdma)

        out_ref[...] = _dot(o_ref[...], wo_ref[pl.ds(p * KBLK, KBLK), :])
        for d in (1, 3, 2):
            j = lax.rem(p + d, N_DEV)
            pltpu.make_async_remote_copy(
                src_ref=o_gat_ref.at[j], dst_ref=o_gat_ref.at[j],
                send_sem=o_rsem.at[j], recv_sem=o_rsem.at[j],
                device_id=(j,), device_id_type=pl.DeviceIdType.MESH,
            ).wait_recv()
            out_ref[...] = out_ref[...] + _dot(
                o_gat_ref[j], wo_ref[pl.ds(j * KBLK, KBLK), :])

        for rdma in o_sends:
            rdma.wait_send()

    out = pl.pallas_call(
        body,
        out_shape=jax.ShapeDtypeStruct((ROWS, D), jnp.float32),
        in_specs=[pl.BlockSpec(memory_space=pltpu.VMEM)] * 8,
        out_specs=pl.BlockSpec(memory_space=pltpu.VMEM),
        scratch_shapes=[
            pltpu.VMEM((ROWS, DC), jnp.float32),
            pltpu.VMEM((N_DEV, ROWS, DC), jnp.float32),
            pltpu.VMEM((N_DEV, DC, KBLK), jnp.float32),
            pltpu.VMEM((N_DEV, DC, KBLK), jnp.float32),
            pltpu.VMEM((ROWS, 2 * KBLK), jnp.float32),
            pltpu.VMEM((ROWS, KBLK), jnp.float32),
            pltpu.VMEM((N_DEV, ROWS, KBLK), jnp.float32),
            pltpu.SemaphoreType.DMA((N_DEV,)),
            pltpu.SemaphoreType.DMA((N_DEV,)),
            pltpu.SemaphoreType.DMA((N_DEV,)),
            pltpu.SemaphoreType.DMA((N_DEV,)),
            pltpu.SemaphoreType.DMA((N_DEV,)),
            pltpu.SemaphoreType.DMA((N_DEV,)),
            pltpu.SemaphoreType.DMA((N_DEV,)),
            pltpu.SemaphoreType.DMA((N_DEV,)),
        ],
        compiler_params=pltpu.CompilerParams(collective_id=0),
    )(x2, Wdkv, Wuk, Wuv, Wq, Wqr, Wkr, Wo)
    return out.reshape(B, S, D)


# device time: 33728 ns/iter; 2.5064x vs baseline; 1.3796x over previous
import jax
import jax.numpy as jnp
from jax import lax
from jax.experimental import pallas as pl
from jax.experimental.pallas import tpu as pltpu

N_DEV = 4
B, S, H, Dh, Dr = 2, 256, 16, 64, 32
D = 1024
DC = 64
ROWS = B * S
HEADS_PER = H // N_DEV
KBLK = HEADS_PER * Dh
QRBLK = HEADS_PER * Dr
SCALE = (Dh + Dr) ** -0.5
BF = jnp.bfloat16


def _dot(a, b):
    return jnp.dot(a, b, preferred_element_type=jnp.float32)


def _dot_t(a, b):
    return lax.dot_general(
        a, b, (((1,), (1,)), ((), ())), preferred_element_type=jnp.float32
    )


def kernel(x, Wdkv, Wuk, Wuv, Wq, Wqr, Wkr, Wo):
    def body(x_ref, wdkv_ref, wuk_ref, wuv_ref, wq_ref, wqr_ref, wkr_ref,
             wo_ref, out_ref,
             c_gat_ref, w_stage_ref, w_gat_ref, o_stage_ref, o_all_ref,
             wo_bf_ref,
             c_ssem, c_rsem, w_ssem, w_rsem, o_ssem, o_rsem):
        p = lax.axis_index("i")

        barrier = pltpu.get_barrier_semaphore()
        for d in range(1, N_DEV):
            nbr = lax.rem(p + d, N_DEV)
            pl.semaphore_signal(
                barrier, inc=1, device_id=(nbr,),
                device_id_type=pl.DeviceIdType.MESH,
            )
        pl.semaphore_wait(barrier, N_DEV - 1)

        for q in range(N_DEV):
            w_stage_ref[q, :, :KBLK] = wuk_ref[:, q * KBLK:(q + 1) * KBLK].astype(BF)
            w_stage_ref[q, :, KBLK:] = wuv_ref[:, q * KBLK:(q + 1) * KBLK].astype(BF)

        sends = []
        for d in (1, 3, 2):
            q = lax.rem(p + d, N_DEV)
            rdma = pltpu.make_async_remote_copy(
                src_ref=w_stage_ref.at[q], dst_ref=w_gat_ref.at[p],
                send_sem=w_ssem.at[q], recv_sem=w_rsem.at[p],
                device_id=(q,), device_id_type=pl.DeviceIdType.MESH,
            )
            rdma.start()
            sends.append(rdma)

        xv = x_ref[...].reshape(ROWS, D).astype(BF)
        c = _dot(xv, wdkv_ref[...].astype(BF))
        cb = c.astype(BF)
        c_gat_ref[p] = cb

        for d in (1, 3, 2):
            q = lax.rem(p + d, N_DEV)
            rdma = pltpu.make_async_remote_copy(
                src_ref=c_gat_ref.at[p], dst_ref=c_gat_ref.at[p],
                send_sem=c_ssem.at[q], recv_sem=c_rsem.at[p],
                device_id=(q,), device_id_type=pl.DeviceIdType.MESH,
            )
            rdma.start()
            sends.append(rdma)
        w_gat_ref[p] = w_stage_ref[p]

        qm = _dot(xv, wq_ref[:, pl.ds(p * KBLK, KBLK)].astype(BF)) * SCALE
        qr = _dot(xv, wqr_ref[:, pl.ds(p * QRBLK, QRBLK)].astype(BF)) * SCALE
        kr = _dot(xv, wkr_ref[...].astype(BF))
        wo_bf_ref[:D // 2, :] = wo_ref[:D // 2, :].astype(BF)

        for d in (1, 3, 2):
            j = lax.rem(p + d, N_DEV)
            for gat, rsem in ((c_gat_ref, c_rsem), (w_gat_ref, w_rsem)):
                pltpu.make_async_remote_copy(
                    src_ref=gat.at[j], dst_ref=gat.at[j],
                    send_sem=rsem.at[j], recv_sem=rsem.at[j],
                    device_id=(j,), device_id_type=pl.DeviceIdType.MESH,
                ).wait_recv()
        c_full = jnp.concatenate(
            [c_gat_ref[j] for j in range(N_DEV)], axis=1)
        w_full = w_gat_ref[...].reshape(N_DEV * DC, 2 * KBLK)
        kv = _dot(c_full, w_full)

        for rdma in sends:
            rdma.wait_send()

        HB = KBLK // 2
        o_sends = []
        for b in range(B):
            r0 = b * S
            krb = kr[r0:r0 + S, :]
            for hh in range(HEADS_PER):
                q = qm[r0:r0 + S, hh * Dh:(hh + 1) * Dh]
                k = kv[r0:r0 + S, hh * Dh:(hh + 1) * Dh]
                v = kv[r0:r0 + S, KBLK + hh * Dh:KBLK + (hh + 1) * Dh]
                qrh = qr[r0:r0 + S, hh * Dr:(hh + 1) * Dr]
                qq = jnp.concatenate([q, qrh], axis=1).astype(BF)
                kk = jnp.concatenate([k, krb], axis=1).astype(BF)
                e = jnp.exp(_dot_t(qq, kk))
                prob = (e / jnp.sum(e, axis=1, keepdims=True)).astype(BF)
                o_stage_ref[b, :, hh * Dh:(hh + 1) * Dh] = _dot(
                    prob, v.astype(BF)).astype(BF)
                if hh % 2 == 1:
                    half = hh // 2
                    f = b * 2 + half
                    lo = half * HB
                    o_all_ref[b, :, pl.ds(p * KBLK + lo, HB)] = (
                        o_stage_ref[b, :, lo:lo + HB])
                    for d in (1, 3, 2):
                        tq = lax.rem(p + d, N_DEV)
                        rdma = pltpu.make_async_remote_copy(
                            src_ref=o_stage_ref.at[b, :, lo:lo + HB],
                            dst_ref=o_all_ref.at[b, :, pl.ds(p * KBLK + lo, HB)],
                            send_sem=o_ssem.at[tq, f], recv_sem=o_rsem.at[p, f],
                            device_id=(tq,), device_id_type=pl.DeviceIdType.MESH,
                        )
                        rdma.start()
                        o_sends.append(rdma)

        wo_bf_ref[D // 2:, :] = wo_ref[D // 2:, :].astype(BF)
        wo_bf = wo_bf_ref[...]
        for b in range(B):
            for half in range(2):
                f = b * 2 + half
                lo = half * HB
                for d in (1, 3, 2):
                    j = lax.rem(p + d, N_DEV)
                    pltpu.make_async_remote_copy(
                        src_ref=o_stage_ref.at[b, :, lo:lo + HB],
                        dst_ref=o_all_ref.at[b, :, pl.ds(j * KBLK + lo, HB)],
                        send_sem=o_rsem.at[j, f], recv_sem=o_rsem.at[j, f],
                        device_id=(j,), device_id_type=pl.DeviceIdType.MESH,
                    ).wait_recv()
            out_ref[b] = _dot(o_all_ref[b], wo_bf)

        for rdma in o_sends:
            rdma.wait_send()

    return pl.pallas_call(
        body,
        out_shape=jax.ShapeDtypeStruct((B, S, D), jnp.float32),
        in_specs=[pl.BlockSpec(memory_space=pltpu.VMEM)] * 8,
        out_specs=pl.BlockSpec(memory_space=pltpu.VMEM),
        scratch_shapes=[
            pltpu.VMEM((N_DEV, ROWS, DC), BF),
            pltpu.VMEM((N_DEV, DC, 2 * KBLK), BF),
            pltpu.VMEM((N_DEV, DC, 2 * KBLK), BF),
            pltpu.VMEM((B, S, KBLK), BF),
            pltpu.VMEM((B, S, D), BF),
            pltpu.VMEM((D, D), BF),
            pltpu.SemaphoreType.DMA((N_DEV,)),
            pltpu.SemaphoreType.DMA((N_DEV,)),
            pltpu.SemaphoreType.DMA((N_DEV,)),
            pltpu.SemaphoreType.DMA((N_DEV,)),
            pltpu.SemaphoreType.DMA((N_DEV, 2 * B)),
            pltpu.SemaphoreType.DMA((N_DEV, 2 * B)),
        ],
        compiler_params=pltpu.CompilerParams(collective_id=0),
    )(x, Wdkv, Wuk, Wuv, Wq, Wqr, Wkr, Wo)
